# baseline (speedup 1.0000x reference)
import functools

import jax
from jax import lax
import jax.numpy as jnp
from jax.experimental import pallas as pl
from jax.experimental.pallas import tpu as pltpu
from jax.experimental.pallas import tpu_sc as plsc

E = 64
TOP_K = 2
H = 768
I = 768
I2 = 2 * I
T = 8192
S = T * TOP_K
BK = 256
NP = S + E * BK
NB = NP // BK
BT = 512

NC = 2
NS = 16
L = 16
CH = S // NS
PPT = NP // (NC * NS)
GCH = 64
TPT = T // (NC * NS)
CT = 32


def _sc_mesh():
    return plsc.VectorSubcoreMesh(core_axis_name="c", subcore_axis_name="s",
                                  num_cores=NC, num_subcores=NS)


def _router_shared_body(x_ref, gw_ref, cb_ref, swi_ref, swo_ref,
                        sh_ref, idx_ref, w_ref):
    xb = x_ref[...]
    h = jnp.dot(xb, swi_ref[...], preferred_element_type=jnp.float32)
    xp = h[:, :I]
    g = h[:, I:]
    act = g * jax.nn.sigmoid(g) * xp
    sh_ref[...] = jnp.dot(act, swo_ref[...], preferred_element_type=jnp.float32)
    logits = jax.nn.sigmoid(
        lax.dot_general(xb, gw_ref[...], (((1,), (1,)), ((), ())),
                        preferred_element_type=jnp.float32))
    sel = logits + cb_ref[...]
    iota = lax.broadcasted_iota(jnp.int32, (BT, E), 1)
    m1 = jnp.max(sel, axis=1, keepdims=True)
    i1 = jnp.min(jnp.where(sel == m1, iota, E), axis=1, keepdims=True)
    w1 = jnp.max(jnp.where(iota == i1, logits, -1.0), axis=1, keepdims=True)
    sel2 = jnp.where(iota == i1, -jnp.inf, sel)
    m2 = jnp.max(sel2, axis=1, keepdims=True)
    i2 = jnp.min(jnp.where(sel2 == m2, iota, E), axis=1, keepdims=True)
    w2 = jnp.max(jnp.where(iota == i2, logits, -1.0), axis=1, keepdims=True)
    ws = w1 + w2
    idx_ref[...] = jnp.concatenate([i1, i2], axis=1)
    w_ref[...] = jnp.concatenate([w1 / ws, w2 / ws], axis=1)


def _router_shared(xf, gate_w, cb2, shared_Wi, shared_Wo):
    return pl.pallas_call(
        _router_shared_body,
        grid=(T // BT,),
        in_specs=[
            pl.BlockSpec((BT, H), lambda b: (b, 0)),
            pl.BlockSpec((E, H), lambda b: (0, 0)),
            pl.BlockSpec((1, E), lambda b: (0, 0)),
            pl.BlockSpec((H, I2), lambda b: (0, 0)),
            pl.BlockSpec((I, H), lambda b: (0, 0)),
        ],
        out_specs=[
            pl.BlockSpec((BT, H), lambda b: (b, 0)),
            pl.BlockSpec((BT, TOP_K), lambda b: (b, 0)),
            pl.BlockSpec((BT, TOP_K), lambda b: (b, 0)),
        ],
        out_shape=[
            jax.ShapeDtypeStruct((T, H), jnp.float32),
            jax.ShapeDtypeStruct((T, TOP_K), jnp.int32),
            jax.ShapeDtypeStruct((T, TOP_K), jnp.float32),
        ],
    )(xf, gate_w, cb2, shared_Wi, shared_Wo)


def _vec_from(fn):
    lanes = lax.iota(jnp.int32, L)
    acc = jnp.zeros((L,), jnp.int32)
    for j in range(L):
        acc = jnp.where(lanes == j, fn(j), acc)
    return acc


@functools.partial(
    pl.kernel,
    out_type=jax.ShapeDtypeStruct((NS, E), jnp.int32),
    mesh=_sc_mesh(),
    compiler_params=pltpu.CompilerParams(needs_layout_passes=False),
    scratch_types=[
        pltpu.VMEM((CH,), jnp.int32),
        pltpu.VMEM((E,), jnp.int32),
        pltpu.SMEM((E,), jnp.int32),
    ],
)
def _hist(eids_hbm, hists_hbm, eids_v, row_v, hist_s):
    c = lax.axis_index("c")
    s = lax.axis_index("s")

    @pl.when(c == 0)
    def _core0():
        pltpu.sync_copy(eids_hbm.at[pl.ds(s * CH, CH)], eids_v)
        for e in range(E):
            hist_s[e] = 0

        def cnt_body(g, carry):
            v = eids_v[pl.ds(g * L, L)]
            for j in range(L):
                e = v[j]
                hist_s[e] = hist_s[e] + 1
            return carry

        lax.fori_loop(0, CH // L, cnt_body, 0)
        for g in range(E // L):
            row_v[pl.ds(g * L, L)] = _vec_from(lambda j: hist_s[g * L + j])
        pltpu.sync_copy(row_v, hists_hbm.at[s])


@functools.partial(
    pl.kernel,
    out_type=[
        jax.ShapeDtypeStruct((S,), jnp.int32),
        jax.ShapeDtypeStruct((NB,), jnp.int32),
    ],
    mesh=_sc_mesh(),
    compiler_params=pltpu.CompilerParams(needs_layout_passes=False),
    scratch_types=[
        pltpu.VMEM((CH,), jnp.int32),
        pltpu.VMEM((NS, E), jnp.int32),
        pltpu.VMEM((NB,), jnp.int32),
        pltpu.VMEM((CH,), jnp.int32),
        pltpu.SMEM((E,), jnp.int32),
        pltpu.SMEM((E,), jnp.int32),
        pltpu.SMEM((E,), jnp.int32),
        pltpu.SMEM((E,), jnp.int32),
        pltpu.SMEM((NB,), jnp.int32),
    ],
)
def _assign(eids_hbm, hists_hbm, pos_hbm, be_hbm,
            eids_v, allh_v, row_v, pos_v,
            cnt_s, base_s, poff_s, rc_s, be_s):
    c = lax.axis_index("c")
    s = lax.axis_index("s")

    @pl.when(c == 0)
    def _core0():
        pltpu.sync_copy(eids_hbm.at[pl.ds(s * CH, CH)], eids_v)
        pltpu.sync_copy(hists_hbm, allh_v)
        for g in range(E // L):
            cnt = jnp.zeros((L,), jnp.int32)
            base = jnp.zeros((L,), jnp.int32)
            for t in range(NS):
                row = allh_v[t, pl.ds(g * L, L)]
                cnt = cnt + row
                base = base + row * (s > t).astype(jnp.int32)
            for j in range(L):
                cnt_s[g * L + j] = cnt[j]
                base_s[g * L + j] = base[j]

        def poff_body(e, acc):
            poff_s[e] = acc
            rc_s[e] = 0
            nbe = (cnt_s[e] + (BK - 1)) // BK
            return acc + nbe * BK

        lax.fori_loop(0, E, poff_body, 0)

        @pl.when(s == 0)
        def _tile0_be():
            def be_outer(e, ptr):
                nbe = (cnt_s[e] + (BK - 1)) // BK

                def be_inner(j, p):
                    be_s[p] = e
                    return p + 1

                return lax.fori_loop(0, nbe, be_inner, ptr)

            nblk = lax.fori_loop(0, E, be_outer, 0)
            last = be_s[nblk - 1]

            def be_fill(b, carry):
                be_s[b] = last
                return carry

            lax.fori_loop(nblk, NB, be_fill, 0)
            for g in range(NB // L):
                row_v[pl.ds(g * L, L)] = _vec_from(lambda j: be_s[g * L + j])
            pltpu.sync_copy(row_v.at[pl.ds(0, NB)], be_hbm)

        def c_body(g, carry):
            v = eids_v[pl.ds(g * L, L)]

            def lane(j):
                e = v[j]
                r = rc_s[e]
                rc_s[e] = r + 1
                return poff_s[e] + base_s[e] + r

            pvec = _vec_from(lane)
            pos_v[pl.ds(g * L, L)] = pvec
            return carry

        lax.fori_loop(0, CH // L, c_body, 0)
        pltpu.sync_copy(pos_v, pos_hbm.at[pl.ds(s * CH, CH)])


@functools.partial(
    pl.kernel,
    out_type=jax.ShapeDtypeStruct((NP, H), jnp.float32),
    mesh=_sc_mesh(),
    compiler_params=pltpu.CompilerParams(needs_layout_passes=False),
    scratch_types=[
        pltpu.VMEM((S,), jnp.int32),
        pltpu.VMEM((PPT,), jnp.int32),
        pltpu.VMEM((GCH,), jnp.int32),
        pltpu.VMEM((GCH,), jnp.int32),
        pltpu.VMEM((GCH, H), jnp.float32),
        pltpu.VMEM((GCH, H), jnp.float32),
        pltpu.SemaphoreType.DMA,
        pltpu.SemaphoreType.DMA,
    ],
)
def _gather(pos_hbm, x_hbm, xs_hbm, pall_v, myst_v,
            idx0_v, idx1_v, rows0_v, rows1_v, sem0, sem1):
    c = lax.axis_index("c")
    s = lax.axis_index("s")
    base = (c * NS + s) * PPT
    zero16 = jnp.zeros((L,), jnp.int32)

    def z_body(i, carry):
        myst_v[pl.ds(i * L, L)] = zero16
        return carry

    lax.fori_loop(0, PPT // L, z_body, 0)
    pltpu.sync_copy(pos_hbm, pall_v)
    lanes = lax.iota(jnp.int32, L)

    def inv_body(i, carry):
        pvec = pall_v[pl.ds(i * L, L)]
        rel = pvec - base
        mask = (rel >= 0) & (rel < PPT)
        rel = jnp.clip(rel, 0, PPT - 1)
        tokvec = (jnp.full((L,), i * L, jnp.int32) + lanes) // TOP_K
        plsc.store_scatter(myst_v, [rel], tokvec, mask=mask)
        return carry

    lax.fori_loop(0, S // L, inv_body, 0)

    idxs = (idx0_v, idx1_v)
    rows = (rows0_v, rows1_v)
    sems = (sem0, sem1)

    def start(g):
        slot = g % 2
        for j in range(GCH // L):
            idxs[slot][pl.ds(j * L, L)] = myst_v[pl.ds(g * GCH + j * L, L)]
        return pltpu.async_copy(x_hbm.at[idxs[slot]], rows[slot], sems[slot])

    nchunks = PPT // GCH
    pend = start(0)
    for g in range(nchunks):
        nxt = start(g + 1) if g + 1 < nchunks else None
        pend.wait()
        pltpu.sync_copy(rows[g % 2], xs_hbm.at[pl.ds(base + g * GCH, GCH)])
        pend = nxt


def _grouped_mm_body(be_ref, xs_ref, wi_ref, wo_ref, ys_ref):
    xb = xs_ref[...]
    h = jnp.dot(xb, wi_ref[0], preferred_element_type=jnp.float32)
    xp = h[:, :I]
    g = h[:, I:]
    act = g * jax.nn.sigmoid(g) * xp
    ys_ref[...] = jnp.dot(act, wo_ref[0], preferred_element_type=jnp.float32)


def _grouped_mm(be, xs, Wi, Wo):
    return pl.pallas_call(
        _grouped_mm_body,
        grid_spec=pltpu.PrefetchScalarGridSpec(
            num_scalar_prefetch=1,
            grid=(NB,),
            in_specs=[
                pl.BlockSpec((BK, H), lambda b, be: (b, 0)),
                pl.BlockSpec((1, H, I2), lambda b, be: (be[b], 0, 0)),
                pl.BlockSpec((1, I, H), lambda b, be: (be[b], 0, 0)),
            ],
            out_specs=pl.BlockSpec((BK, H), lambda b, be: (b, 0)),
        ),
        out_shape=jax.ShapeDtypeStruct((NP, H), jnp.float32),
    )(be, xs, Wi, Wo)


@functools.partial(
    pl.kernel,
    out_type=jax.ShapeDtypeStruct((T, H), jnp.float32),
    mesh=_sc_mesh(),
    compiler_params=pltpu.CompilerParams(needs_layout_passes=False),
    scratch_types=[
        pltpu.VMEM((CT * TOP_K,), jnp.int32),
        pltpu.VMEM((CT * TOP_K,), jnp.float32),
        pltpu.VMEM((CT * TOP_K, H), jnp.float32),
        pltpu.VMEM((CT, H), jnp.float32),
        pltpu.VMEM((CT, H), jnp.float32),
        pltpu.SemaphoreType.DMA,
    ],
)
def _combine(pos_hbm, wf_hbm, ys_hbm, sh_hbm, out_hbm,
             pidx_v, wv_v, rows_v, sh_v, out_v, sem):
    c = lax.axis_index("c")
    s = lax.axis_index("s")
    tbase = (c * NS + s) * TPT

    def chunk(g, carry):
        t0 = tbase + g * CT
        pltpu.sync_copy(pos_hbm.at[pl.ds(t0 * TOP_K, CT * TOP_K)], pidx_v)
        pltpu.sync_copy(wf_hbm.at[pl.ds(t0 * TOP_K, CT * TOP_K)], wv_v)
        pltpu.async_copy(ys_hbm.at[pidx_v], rows_v, sem).wait()
        pltpu.sync_copy(sh_hbm.at[pl.ds(t0, CT)], sh_v)

        def tok(gg, carry2):
            wvec = wv_v[pl.ds(gg * L, L)]
            for j in range(L // TOP_K):
                t = gg * (L // TOP_K) + j
                w0 = wvec[2 * j]
                w1 = wvec[2 * j + 1]
                for l in range(H // L):
                    sl = pl.ds(l * L, L)
                    out_v[t, sl] = (w0 * rows_v[2 * t, sl]
                                    + w1 * rows_v[2 * t + 1, sl] + sh_v[t, sl])
            return carry2

        lax.fori_loop(0, CT * TOP_K // L, tok, 0)
        pltpu.sync_copy(out_v, out_hbm.at[pl.ds(t0, CT)])
        return carry

    lax.fori_loop(0, TPT // CT, chunk, 0)


def kernel(x, gate_w, correction_bias, Wi, Wo, shared_Wi, shared_Wo):
    orig_shape = x.shape
    xf = x.reshape(-1, H)
    cb2 = correction_bias.reshape(1, E)

    shared_out, idx, w = _router_shared(xf, gate_w, cb2, shared_Wi, shared_Wo)
    eids = idx.reshape(-1)
    hists = _hist(eids)
    pos, be = _assign(eids, hists)
    xs = _gather(pos, xf)
    ys = _grouped_mm(be, xs, Wi, Wo)
    out = _combine(pos, w.reshape(-1), ys, shared_out)
    return out.reshape(orig_shape)


# speedup vs baseline: 1.3432x; 1.3432x over previous
import functools

import jax
from jax import lax
import jax.numpy as jnp
from jax.experimental import pallas as pl
from jax.experimental.pallas import tpu as pltpu
from jax.experimental.pallas import tpu_sc as plsc

E = 64
TOP_K = 2
H = 768
I = 768
I2 = 2 * I
T = 8192
S = T * TOP_K
BK = 256
NP = S + E * BK
NB = NP // BK
BT = 512

NC = 2
NS = 16
L = 16
CH = S // NS
PPT = NP // (NC * NS)
GCH = 64
TPT = T // (NC * NS)
CT = 32


def _sc_mesh():
    return plsc.VectorSubcoreMesh(core_axis_name="c", subcore_axis_name="s",
                                  num_cores=NC, num_subcores=NS)


def _router_shared_body(x_ref, gw_ref, cb_ref, swi_ref, swo_ref,
                        sh_ref, idx_ref, w_ref):
    xb = x_ref[...]
    h = jnp.dot(xb, swi_ref[...], preferred_element_type=jnp.float32)
    xp = h[:, :I]
    g = h[:, I:]
    act = g * jax.nn.sigmoid(g) * xp
    sh_ref[...] = jnp.dot(act, swo_ref[...], preferred_element_type=jnp.float32)
    logits = jax.nn.sigmoid(
        lax.dot_general(xb, gw_ref[...], (((1,), (1,)), ((), ())),
                        preferred_element_type=jnp.float32))
    sel = logits + cb_ref[...]
    iota = lax.broadcasted_iota(jnp.int32, (BT, E), 1)
    m1 = jnp.max(sel, axis=1, keepdims=True)
    i1 = jnp.min(jnp.where(sel == m1, iota, E), axis=1, keepdims=True)
    w1 = jnp.max(jnp.where(iota == i1, logits, -1.0), axis=1, keepdims=True)
    sel2 = jnp.where(iota == i1, -jnp.inf, sel)
    m2 = jnp.max(sel2, axis=1, keepdims=True)
    i2 = jnp.min(jnp.where(sel2 == m2, iota, E), axis=1, keepdims=True)
    w2 = jnp.max(jnp.where(iota == i2, logits, -1.0), axis=1, keepdims=True)
    ws = w1 + w2
    idx_ref[...] = jnp.concatenate([i1, i2], axis=1)
    w_ref[...] = jnp.concatenate([w1 / ws, w2 / ws], axis=1)


def _router_shared(xf, gate_w, cb2, shared_Wi, shared_Wo):
    return pl.pallas_call(
        _router_shared_body,
        grid=(T // BT,),
        in_specs=[
            pl.BlockSpec((BT, H), lambda b: (b, 0)),
            pl.BlockSpec((E, H), lambda b: (0, 0)),
            pl.BlockSpec((1, E), lambda b: (0, 0)),
            pl.BlockSpec((H, I2), lambda b: (0, 0)),
            pl.BlockSpec((I, H), lambda b: (0, 0)),
        ],
        out_specs=[
            pl.BlockSpec((BT, H), lambda b: (b, 0)),
            pl.BlockSpec((BT, TOP_K), lambda b: (b, 0)),
            pl.BlockSpec((BT, TOP_K), lambda b: (b, 0)),
        ],
        out_shape=[
            jax.ShapeDtypeStruct((T, H), jnp.float32),
            jax.ShapeDtypeStruct((T, TOP_K), jnp.int32),
            jax.ShapeDtypeStruct((T, TOP_K), jnp.float32),
        ],
    )(xf, gate_w, cb2, shared_Wi, shared_Wo)


def _vec_from(fn):
    lanes = lax.iota(jnp.int32, L)
    acc = jnp.zeros((L,), jnp.int32)
    for j in range(L):
        acc = jnp.where(lanes == j, fn(j), acc)
    return acc


@functools.partial(
    pl.kernel,
    out_type=jax.ShapeDtypeStruct((NS, E), jnp.int32),
    mesh=_sc_mesh(),
    compiler_params=pltpu.CompilerParams(needs_layout_passes=False),
    scratch_types=[
        pltpu.VMEM((CH,), jnp.int32),
        pltpu.VMEM((E,), jnp.int32),
        pltpu.SMEM((E,), jnp.int32),
    ],
)
def _hist(eids_hbm, hists_hbm, eids_v, row_v, hist_s):
    c = lax.axis_index("c")
    s = lax.axis_index("s")

    @pl.when(c == 0)
    def _core0():
        pltpu.sync_copy(eids_hbm.at[pl.ds(s * CH, CH)], eids_v)
        for e in range(E):
            hist_s[e] = 0

        def cnt_body(g, carry):
            v = eids_v[pl.ds(g * L, L)]
            for j in range(L):
                e = v[j]
                hist_s[e] = hist_s[e] + 1
            return carry

        lax.fori_loop(0, CH // L, cnt_body, 0)
        for g in range(E // L):
            row_v[pl.ds(g * L, L)] = _vec_from(lambda j: hist_s[g * L + j])
        pltpu.sync_copy(row_v, hists_hbm.at[s])


@functools.partial(
    pl.kernel,
    out_type=[
        jax.ShapeDtypeStruct((S,), jnp.int32),
        jax.ShapeDtypeStruct((NB,), jnp.int32),
        jax.ShapeDtypeStruct((L,), jnp.int32),
    ],
    mesh=_sc_mesh(),
    compiler_params=pltpu.CompilerParams(needs_layout_passes=False),
    scratch_types=[
        pltpu.VMEM((CH,), jnp.int32),
        pltpu.VMEM((NS, E), jnp.int32),
        pltpu.VMEM((NB,), jnp.int32),
        pltpu.VMEM((CH,), jnp.int32),
        pltpu.VMEM((L,), jnp.int32),
        pltpu.SMEM((E,), jnp.int32),
        pltpu.SMEM((E,), jnp.int32),
        pltpu.SMEM((E,), jnp.int32),
        pltpu.SMEM((E,), jnp.int32),
        pltpu.SMEM((NB,), jnp.int32),
    ],
)
def _assign(eids_hbm, hists_hbm, pos_hbm, be_hbm, npad_hbm,
            eids_v, allh_v, row_v, pos_v, npv_v,
            cnt_s, base_s, poff_s, rc_s, be_s):
    c = lax.axis_index("c")
    s = lax.axis_index("s")

    @pl.when(c == 0)
    def _core0():
        pltpu.sync_copy(eids_hbm.at[pl.ds(s * CH, CH)], eids_v)
        pltpu.sync_copy(hists_hbm, allh_v)
        for g in range(E // L):
            cnt = jnp.zeros((L,), jnp.int32)
            base = jnp.zeros((L,), jnp.int32)
            for t in range(NS):
                row = allh_v[t, pl.ds(g * L, L)]
                cnt = cnt + row
                base = base + row * (s > t).astype(jnp.int32)
            for j in range(L):
                cnt_s[g * L + j] = cnt[j]
                base_s[g * L + j] = base[j]

        def poff_body(e, acc):
            poff_s[e] = acc
            rc_s[e] = 0
            nbe = (cnt_s[e] + (BK - 1)) // BK
            return acc + nbe * BK

        npad = lax.fori_loop(0, E, poff_body, 0)

        @pl.when(s == 0)
        def _npad_out():
            npv_v[...] = jnp.where(lax.iota(jnp.int32, L) == 0, npad, 0)
            pltpu.sync_copy(npv_v, npad_hbm)

        @pl.when(s == 0)
        def _tile0_be():
            def be_outer(e, ptr):
                nbe = (cnt_s[e] + (BK - 1)) // BK

                def be_inner(j, p):
                    be_s[p] = e
                    return p + 1

                return lax.fori_loop(0, nbe, be_inner, ptr)

            nblk = lax.fori_loop(0, E, be_outer, 0)
            last = be_s[nblk - 1]

            def be_fill(b, carry):
                be_s[b] = last
                return carry

            lax.fori_loop(nblk, NB, be_fill, 0)
            for g in range(NB // L):
                row_v[pl.ds(g * L, L)] = _vec_from(lambda j: be_s[g * L + j])
            pltpu.sync_copy(row_v.at[pl.ds(0, NB)], be_hbm)

        def c_body(g, carry):
            v = eids_v[pl.ds(g * L, L)]

            def lane(j):
                e = v[j]
                r = rc_s[e]
                rc_s[e] = r + 1
                return poff_s[e] + base_s[e] + r

            pvec = _vec_from(lane)
            pos_v[pl.ds(g * L, L)] = pvec
            return carry

        lax.fori_loop(0, CH // L, c_body, 0)
        pltpu.sync_copy(pos_v, pos_hbm.at[pl.ds(s * CH, CH)])


@functools.partial(
    pl.kernel,
    out_type=jax.ShapeDtypeStruct((NP, H), jnp.float32),
    mesh=_sc_mesh(),
    compiler_params=pltpu.CompilerParams(needs_layout_passes=False),
    scratch_types=[
        pltpu.VMEM((S,), jnp.int32),
        pltpu.VMEM((PPT,), jnp.int32),
        pltpu.VMEM((L,), jnp.int32),
        pltpu.VMEM((GCH,), jnp.int32),
        pltpu.VMEM((GCH,), jnp.int32),
        pltpu.VMEM((GCH, H), jnp.float32),
        pltpu.VMEM((GCH, H), jnp.float32),
        pltpu.SemaphoreType.DMA,
        pltpu.SemaphoreType.DMA,
    ],
)
def _gather(pos_hbm, x_hbm, npad_hbm, xs_hbm, pall_v, myst_v, npv_v,
            idx0_v, idx1_v, rows0_v, rows1_v, sem0, sem1):
    c = lax.axis_index("c")
    s = lax.axis_index("s")
    base = (c * NS + s) * PPT
    zero16 = jnp.zeros((L,), jnp.int32)
    pltpu.sync_copy(npad_hbm, npv_v)
    npr = npv_v[pl.ds(0, L)][0]

    def z_body(i, carry):
        myst_v[pl.ds(i * L, L)] = zero16
        return carry

    lax.fori_loop(0, PPT // L, z_body, 0)
    pltpu.sync_copy(pos_hbm, pall_v)
    lanes = lax.iota(jnp.int32, L)

    def inv_body(i, carry):
        pvec = pall_v[pl.ds(i * L, L)]
        rel = pvec - base
        mask = (rel >= 0) & (rel < PPT)
        rel = jnp.clip(rel, 0, PPT - 1)
        tokvec = (jnp.full((L,), i * L, jnp.int32) + lanes) // TOP_K
        plsc.store_scatter(myst_v, [rel], tokvec, mask=mask)
        return carry

    lax.fori_loop(0, S // L, inv_body, 0)

    idxs = (idx0_v, idx1_v)
    rows = (rows0_v, rows1_v)
    sems = (sem0, sem1)

    for g in range(PPT // GCH):
        @pl.when(base + g * GCH < npr)
        def _do(g=g):
            slot = g % 2
            for j in range(GCH // L):
                idxs[slot][pl.ds(j * L, L)] = myst_v[pl.ds(g * GCH + j * L, L)]
            pltpu.async_copy(x_hbm.at[idxs[slot]], rows[slot], sems[slot]).wait()
            pltpu.sync_copy(rows[slot], xs_hbm.at[pl.ds(base + g * GCH, GCH)])


def _grouped_mm_body(be_ref, nb_ref, xs_ref, wi_ref, wo_ref, ys_ref):
    @pl.when(pl.program_id(0) < nb_ref[0])
    def _live():
        xb = xs_ref[...]
        h = jnp.dot(xb, wi_ref[0], preferred_element_type=jnp.float32)
        xp = h[:, :I]
        g = h[:, I:]
        act = g * jax.nn.sigmoid(g) * xp
        ys_ref[...] = jnp.dot(act, wo_ref[0],
                              preferred_element_type=jnp.float32)


def _grouped_mm(be, nbv, xs, Wi, Wo):
    return pl.pallas_call(
        _grouped_mm_body,
        grid_spec=pltpu.PrefetchScalarGridSpec(
            num_scalar_prefetch=2,
            grid=(NB,),
            in_specs=[
                pl.BlockSpec((BK, H),
                             lambda b, be, nb: (jnp.where(b < nb[0], b, 0), 0)),
                pl.BlockSpec((1, H, I2), lambda b, be, nb: (be[b], 0, 0)),
                pl.BlockSpec((1, I, H), lambda b, be, nb: (be[b], 0, 0)),
            ],
            out_specs=pl.BlockSpec((BK, H), lambda b, be, nb: (b, 0)),
        ),
        out_shape=jax.ShapeDtypeStruct((NP, H), jnp.float32),
    )(be, nbv, xs, Wi, Wo)


@functools.partial(
    pl.kernel,
    out_type=jax.ShapeDtypeStruct((T, H), jnp.float32),
    mesh=_sc_mesh(),
    compiler_params=pltpu.CompilerParams(needs_layout_passes=False),
    scratch_types=[
        pltpu.VMEM((CT * TOP_K,), jnp.int32),
        pltpu.VMEM((CT * TOP_K,), jnp.float32),
        pltpu.VMEM((CT * TOP_K, H), jnp.float32),
        pltpu.VMEM((CT, H), jnp.float32),
        pltpu.VMEM((CT, H), jnp.float32),
        pltpu.SemaphoreType.DMA,
    ],
)
def _combine(pos_hbm, wf_hbm, ys_hbm, sh_hbm, out_hbm,
             pidx_v, wv_v, rows_v, sh_v, out_v, sem):
    c = lax.axis_index("c")
    s = lax.axis_index("s")
    tbase = (c * NS + s) * TPT

    def chunk(g, carry):
        t0 = tbase + g * CT
        pltpu.sync_copy(pos_hbm.at[pl.ds(t0 * TOP_K, CT * TOP_K)], pidx_v)
        pltpu.sync_copy(wf_hbm.at[pl.ds(t0 * TOP_K, CT * TOP_K)], wv_v)
        pltpu.async_copy(ys_hbm.at[pidx_v], rows_v, sem).wait()
        pltpu.sync_copy(sh_hbm.at[pl.ds(t0, CT)], sh_v)

        def tok(gg, carry2):
            wvec = wv_v[pl.ds(gg * L, L)]
            for j in range(L // TOP_K):
                t = gg * (L // TOP_K) + j
                w0 = wvec[2 * j]
                w1 = wvec[2 * j + 1]
                for l in range(H // L):
                    sl = pl.ds(l * L, L)
                    out_v[t, sl] = (w0 * rows_v[2 * t, sl]
                                    + w1 * rows_v[2 * t + 1, sl] + sh_v[t, sl])
            return carry2

        lax.fori_loop(0, CT * TOP_K // L, tok, 0)
        pltpu.sync_copy(out_v, out_hbm.at[pl.ds(t0, CT)])
        return carry

    lax.fori_loop(0, TPT // CT, chunk, 0)


def kernel(x, gate_w, correction_bias, Wi, Wo, shared_Wi, shared_Wo):
    orig_shape = x.shape
    xf = x.reshape(-1, H)
    cb2 = correction_bias.reshape(1, E)

    shared_out, idx, w = _router_shared(xf, gate_w, cb2, shared_Wi, shared_Wo)
    eids = idx.reshape(-1)
    hists = _hist(eids)
    pos, be, npad = _assign(eids, hists)
    xs = _gather(pos, xf, npad)
    nbv = npad[0:1] // BK
    ys = _grouped_mm(be, nbv, xs, Wi, Wo)
    out = _combine(pos, w.reshape(-1), ys, shared_out)
    return out.reshape(orig_shape)
